# baseline (speedup 1.0000x reference)
import functools

import jax
import jax.numpy as jnp
from jax import lax
from jax.experimental import pallas as pl
from jax.experimental.pallas import tpu as pltpu
from jax.experimental.pallas import tpu_sc as plsc

N = 10000
E = 320000
D = 128
H = 128

NC = 2
NS = 16
LANES = 16
NW = NC * NS
NP = 10240
SPAN = NP // NS
ROWS = 2560
RPW = ROWS // NW
RPT = ROWS // NS
PAD_IDX = N

NA = 2512
DISCARD = 2504
HSPAN = 632
ASPAN = 160


def _mesh():
    return plsc.VectorSubcoreMesh(core_axis_name="c", subcore_axis_name="s")


@functools.partial(
    pl.kernel,
    mesh=_mesh(),
    out_type=jax.ShapeDtypeStruct((NC, 2, NP), jnp.float32),
    scratch_types=[
        pltpu.VMEM((RPW, 128), jnp.int32),
        pltpu.VMEM((RPW, 128), jnp.int32),
        pltpu.VMEM((128,), jnp.float32),
        pltpu.VMEM_SHARED((NP,), jnp.float32),
        pltpu.VMEM_SHARED((NP,), jnp.float32),
    ],
)
def _sc_degrees(src_hbm, dst_hbm, zer_hbm, out_hbm,
                sidx, didx, ones_v, shist, dhist):
    c = lax.axis_index("c")
    s = lax.axis_index("s")
    w = s * NC + c
    for i in range(128 // LANES):
        ones_v[pl.ds(i * LANES, LANES)] = jnp.ones((LANES,), jnp.float32)
    pltpu.sync_copy(zer_hbm, shist.at[pl.ds(s * SPAN, SPAN)])
    pltpu.sync_copy(zer_hbm, dhist.at[pl.ds(s * SPAN, SPAN)])
    pltpu.sync_copy(src_hbm.at[pl.ds(w * RPW, RPW)], sidx)
    pltpu.sync_copy(dst_hbm.at[pl.ds(w * RPW, RPW)], didx)
    plsc.subcore_barrier()

    def step(j, carry):
        pltpu.sync_copy(ones_v, shist.at[sidx.at[j]], add=True)
        pltpu.sync_copy(ones_v, dhist.at[didx.at[j]], add=True)
        return carry

    lax.fori_loop(0, RPW, step, 0)
    plsc.subcore_barrier()
    pltpu.sync_copy(shist.at[pl.ds(s * SPAN, SPAN)],
                    out_hbm.at[c, 0, pl.ds(s * SPAN, SPAN)])
    pltpu.sync_copy(dhist.at[pl.ds(s * SPAN, SPAN)],
                    out_hbm.at[c, 1, pl.ds(s * SPAN, SPAN)])


@functools.partial(
    pl.kernel,
    mesh=_mesh(),
    out_type=jax.ShapeDtypeStruct((NC, 2, NA, 128), jnp.float32),
    scratch_types=[
        pltpu.VMEM((8, 128), jnp.int32),
        pltpu.VMEM((16, 64), jnp.int32),
        pltpu.VMEM((64, 128), jnp.float32),
        pltpu.VMEM((64, 128), jnp.float32),
        pltpu.VMEM_SHARED((N, 128), jnp.float32),
        pltpu.VMEM_SHARED((NA, 128), jnp.float32),
        pltpu.SemaphoreType.DMA,
        pltpu.SemaphoreType.DMA,
    ],
)
def _sc_agg(h_hbm, src_hbm, dstl_hbm, zer_hbm, out_hbm,
            sidx, didx, gb0, gb1, h_sh, agg_sh, sm0, sm1):
    c = lax.axis_index("c")
    s = lax.axis_index("s")
    gbufs = (gb0, gb1)
    sems = (sm0, sm1)

    @pl.when(s < NS - 1)
    def _():
        pltpu.sync_copy(h_hbm.at[pl.ds(s * HSPAN, HSPAN)],
                        h_sh.at[pl.ds(s * HSPAN, HSPAN)])

    @pl.when(s == NS - 1)
    def _():
        pltpu.sync_copy(h_hbm.at[pl.ds((NS - 1) * HSPAN, N - (NS - 1) * HSPAN)],
                        h_sh.at[pl.ds((NS - 1) * HSPAN, N - (NS - 1) * HSPAN)])

    r0 = s * RPT

    for k in range(2):
        @pl.when(s < NS - 1)
        def _():
            pltpu.sync_copy(zer_hbm.at[pl.ds(0, ASPAN)],
                            agg_sh.at[pl.ds(s * ASPAN, ASPAN)])

        @pl.when(s == NS - 1)
        def _():
            pltpu.sync_copy(
                zer_hbm.at[pl.ds(0, NA - (NS - 1) * ASPAN)],
                agg_sh.at[pl.ds((NS - 1) * ASPAN, NA - (NS - 1) * ASPAN)])

        plsc.subcore_barrier()

        def block(b, carry):
            row = r0 + b * 8
            pltpu.sync_copy(src_hbm.at[pl.ds(row, 8)], sidx)
            pltpu.sync_copy(dstl_hbm.at[c, k, pl.ds(row * 2, 16)], didx)
            pltpu.async_copy(h_sh.at[sidx.at[0, pl.ds(0, 64)]], gb0, sm0)
            pltpu.async_copy(h_sh.at[sidx.at[0, pl.ds(64, 64)]], gb1, sm1)

            def qstep(qq, qcarry):
                for u in range(2):
                    bu = gbufs[u]
                    su = sems[u]
                    t = qq * 2 + u
                    pltpu.make_async_copy(
                        h_sh.at[sidx.at[qq, pl.ds(u * 64, 64)]], bu, su).wait()
                    pltpu.sync_copy(bu, agg_sh.at[didx.at[t]], add=True)

                    @pl.when(t + 2 < 16)
                    def _():
                        pltpu.async_copy(
                            h_sh.at[sidx.at[qq + 1, pl.ds(u * 64, 64)]],
                            bu, su)
                return qcarry

            lax.fori_loop(0, 8, qstep, 0)
            return carry

        lax.fori_loop(0, RPT // 8, block, 0)
        plsc.subcore_barrier()

        @pl.when(s < NS - 1)
        def _():
            pltpu.sync_copy(agg_sh.at[pl.ds(s * ASPAN, ASPAN)],
                            out_hbm.at[c, k, pl.ds(s * ASPAN, ASPAN)])

        @pl.when(s == NS - 1)
        def _():
            pltpu.sync_copy(
                agg_sh.at[pl.ds((NS - 1) * ASPAN, NA - (NS - 1) * ASPAN)],
                out_hbm.at[c, k, pl.ds((NS - 1) * ASPAN, NA - (NS - 1) * ASPAN)])

        plsc.subcore_barrier()


def _tc_norm_body(x_ref, degs_ref, dst_ref, h_ref, dstl_ref):
    x = x_ref[...]
    nrm = jnp.sqrt(jnp.sum(x * x, axis=1, keepdims=True))
    xn = x / jnp.maximum(nrm, 1e-12)
    deg_out = degs_ref[:, 0:1] + degs_ref[:, 2:3]
    ns = lax.rsqrt(jnp.maximum(deg_out, 1.0))
    h_ref[...] = xn * ns
    dst = dst_ref[...]
    blk = lax.shift_right_logical(dst, 3)
    loc = jnp.bitwise_or(lax.shift_left(lax.shift_right_logical(blk, 2), 3),
                         jnp.bitwise_and(dst, 7))
    q = jnp.bitwise_and(blk, 3)
    for cc in range(2):
        for kk in range(2):
            dstl_ref[cc, kk] = jnp.where(q == 2 * kk + cc, loc, DISCARD)


def _tc_out_body(x_ref, degs_ref, agg_ref, id_ref,
                 wv_ref, bv_ref, wt_ref, bt_ref, out_ref):
    x = x_ref[...]
    nrm = jnp.sqrt(jnp.sum(x * x, axis=1, keepdims=True))
    xn = x / jnp.maximum(nrm, 1e-12)
    deg_in = degs_ref[:, 1:2] + degs_ref[:, 3:4]
    nd = lax.rsqrt(jnp.maximum(deg_in, 1.0))
    agg = agg_ref[...] * nd
    w = (wv_ref[...] + wt_ref[...]) * 0.5
    b = (bv_ref[...] + bt_ref[...]) * 0.5
    out2 = (jnp.dot(agg, w, preferred_element_type=jnp.float32,
                    precision=lax.Precision.HIGHEST)
            + b[None, :] + id_ref[...])
    out_ref[:, :D] = xn
    out_ref[:, D:] = out2


def kernel(x, edge_index, id_embedding, W_v, b_v, W_t, b_t):
    npad = ROWS * 128 - E
    pad_dst = jnp.full((npad,), PAD_IDX, jnp.int32)
    src_deg = jnp.concatenate([edge_index[0], pad_dst]).reshape(ROWS, 128)
    dst_deg = jnp.concatenate([edge_index[1], pad_dst]).reshape(ROWS, 128)
    src_agg = jnp.concatenate(
        [edge_index[0], jnp.zeros((npad,), jnp.int32)]).reshape(ROWS, 128)
    zer1 = jnp.zeros((SPAN,), jnp.float32)
    zer2 = jnp.zeros((ASPAN, 128), jnp.float32)

    degs_raw = _sc_degrees(src_deg, dst_deg, zer1)
    degs = jnp.transpose(degs_raw.reshape(2 * NC, NP))[:N]

    h, dstl = pl.pallas_call(
        _tc_norm_body,
        out_shape=(jax.ShapeDtypeStruct((N, D), jnp.float32),
                   jax.ShapeDtypeStruct((NC, 2, ROWS, 128), jnp.int32)),
    )(x, degs, dst_deg)
    dstl64 = dstl.reshape(NC, 2, ROWS * 2, 64)

    aggs = _sc_agg(h, src_agg, dstl64, zer2)
    nb = DISCARD // 8
    agg = jnp.stack([aggs[0, 0, :DISCARD].reshape(nb, 8, 128),
                     aggs[1, 0, :DISCARD].reshape(nb, 8, 128),
                     aggs[0, 1, :DISCARD].reshape(nb, 8, 128),
                     aggs[1, 1, :DISCARD].reshape(nb, 8, 128)],
                    axis=1).reshape(nb * 32, 128)[:N]

    out = pl.pallas_call(
        _tc_out_body,
        out_shape=jax.ShapeDtypeStruct((N, D + H), jnp.float32),
    )(x, degs, agg, id_embedding, W_v, b_v, W_t, b_t)
    return out


# speedup vs baseline: 1.4550x; 1.4550x over previous
import functools

import jax
import jax.numpy as jnp
from jax import lax
from jax.experimental import pallas as pl
from jax.experimental.pallas import tpu as pltpu
from jax.experimental.pallas import tpu_sc as plsc

N = 10000
E = 320000
D = 128
H = 128

NC = 2
NS = 16
LANES = 16
NW = NC * NS
NP = 10240
ROWS = 2560
RPW = ROWS // NW
SPAN = NP // NS
PAD_IDX = N


def _mesh():
    return plsc.VectorSubcoreMesh(core_axis_name="c", subcore_axis_name="s")


@functools.partial(
    pl.kernel,
    mesh=_mesh(),
    out_type=jax.ShapeDtypeStruct((NC, 2, NP), jnp.float32),
    scratch_types=[
        pltpu.VMEM((RPW, 128), jnp.int32),
        pltpu.VMEM((RPW, 128), jnp.int32),
        pltpu.VMEM((128,), jnp.float32),
        pltpu.VMEM_SHARED((NP,), jnp.float32),
        pltpu.VMEM_SHARED((NP,), jnp.float32),
    ],
)
def _sc_degrees(src_hbm, dst_hbm, zer_hbm, out_hbm,
                sidx, didx, ones_v, shist, dhist):
    c = lax.axis_index("c")
    s = lax.axis_index("s")
    w = s * NC + c
    for i in range(128 // LANES):
        ones_v[pl.ds(i * LANES, LANES)] = jnp.ones((LANES,), jnp.float32)
    pltpu.sync_copy(zer_hbm, shist.at[pl.ds(s * SPAN, SPAN)])
    pltpu.sync_copy(zer_hbm, dhist.at[pl.ds(s * SPAN, SPAN)])
    pltpu.sync_copy(src_hbm.at[pl.ds(w * RPW, RPW)], sidx)
    pltpu.sync_copy(dst_hbm.at[pl.ds(w * RPW, RPW)], didx)
    plsc.subcore_barrier()

    def step(j, carry):
        pltpu.sync_copy(ones_v, shist.at[sidx.at[j]], add=True)
        pltpu.sync_copy(ones_v, dhist.at[didx.at[j]], add=True)
        return carry

    lax.fori_loop(0, RPW, step, 0)
    plsc.subcore_barrier()
    pltpu.sync_copy(shist.at[pl.ds(s * SPAN, SPAN)],
                    out_hbm.at[c, 0, pl.ds(s * SPAN, SPAN)])
    pltpu.sync_copy(dhist.at[pl.ds(s * SPAN, SPAN)],
                    out_hbm.at[c, 1, pl.ds(s * SPAN, SPAN)])


@functools.partial(
    pl.kernel,
    mesh=_mesh(),
    out_type=jax.ShapeDtypeStruct((NC, NP, 128), jnp.float32),
    scratch_types=[
        pltpu.VMEM((RPW // 2, 128), jnp.int32),
        pltpu.VMEM((RPW, 64), jnp.int32),
        pltpu.VMEM((64, 128), jnp.float32),
        pltpu.VMEM((64, 128), jnp.float32),
        pltpu.VMEM((64, 128), jnp.float32),
        pltpu.VMEM((64, 128), jnp.float32),
        pltpu.VMEM_SHARED((NP, 128), jnp.float32),
        pltpu.SemaphoreType.DMA,
        pltpu.SemaphoreType.DMA,
        pltpu.SemaphoreType.DMA,
        pltpu.SemaphoreType.DMA,
    ],
)
def _sc_agg(h_hbm, src_hbm, dst64_hbm, zer_hbm, out_hbm,
            sidx, didx, gb0, gb1, gb2, gb3, agg_sh, sm0, sm1, sm2, sm3):
    c = lax.axis_index("c")
    s = lax.axis_index("s")
    w = s * NC + c
    half = RPW // 2
    nsub = RPW
    gbufs = (gb0, gb1, gb2, gb3)
    sems = (sm0, sm1, sm2, sm3)
    pltpu.sync_copy(zer_hbm, agg_sh.at[pl.ds(s * SPAN, SPAN)])
    plsc.subcore_barrier()

    def sidx_at(j, hlf):
        return sidx.at[j, pl.ds(hlf * 64, 64)]

    for p in range(2):
        pltpu.sync_copy(src_hbm.at[pl.ds(w * RPW + p * half, half)], sidx)
        pltpu.sync_copy(dst64_hbm.at[pl.ds((w * RPW + p * half) * 2, nsub)],
                        didx)
        pltpu.async_copy(h_hbm.at[sidx_at(0, 0)], gb0, sm0)
        pltpu.async_copy(h_hbm.at[sidx_at(0, 1)], gb1, sm1)

        def step(i, carry):
            t0 = i * 4
            for u in range(4):
                t = t0 + u
                j = lax.div(t, 2)
                jn = lax.div(t + 2, 2)
                pltpu.make_async_copy(
                    h_hbm.at[sidx_at(j, u % 2)], gbufs[u], sems[u]).wait()

                @pl.when(t + 2 < nsub)
                def _():
                    pltpu.async_copy(h_hbm.at[sidx_at(jn, u % 2)],
                                     gbufs[(u + 2) % 4], sems[(u + 2) % 4])

                pltpu.sync_copy(gbufs[u], agg_sh.at[didx.at[t]], add=True)
            return carry

        lax.fori_loop(0, nsub // 4, step, 0)
    plsc.subcore_barrier()
    pltpu.sync_copy(agg_sh.at[pl.ds(s * SPAN, SPAN)],
                    out_hbm.at[c, pl.ds(s * SPAN, SPAN)])


def _tc_norm_body(x_ref, degs_ref, h_ref):
    x = x_ref[...]
    nrm = jnp.sqrt(jnp.sum(x * x, axis=1, keepdims=True))
    xn = x / jnp.maximum(nrm, 1e-12)
    deg_out = degs_ref[:, 0:1] + degs_ref[:, 2:3]
    ns = lax.rsqrt(jnp.maximum(deg_out, 1.0))
    h_ref[...] = xn * ns


def _tc_out_body(x_ref, degs_ref, aggs_ref, id_ref,
                 wv_ref, bv_ref, wt_ref, bt_ref, out_ref):
    x = x_ref[...]
    nrm = jnp.sqrt(jnp.sum(x * x, axis=1, keepdims=True))
    xn = x / jnp.maximum(nrm, 1e-12)
    deg_in = degs_ref[:, 1:2] + degs_ref[:, 3:4]
    nd = lax.rsqrt(jnp.maximum(deg_in, 1.0))
    agg = (aggs_ref[0] + aggs_ref[1]) * nd
    w = (wv_ref[...] + wt_ref[...]) * 0.5
    b = (bv_ref[...] + bt_ref[...]) * 0.5
    out2 = (jnp.dot(agg, w, preferred_element_type=jnp.float32,
                    precision=lax.Precision.HIGHEST)
            + b[None, :] + id_ref[...])
    out_ref[:, :D] = xn
    out_ref[:, D:] = out2


def kernel(x, edge_index, id_embedding, W_v, b_v, W_t, b_t):
    pad = jnp.full((ROWS * 128 - E,), PAD_IDX, jnp.int32)
    src = jnp.concatenate([edge_index[0], pad]).reshape(ROWS, 128)
    dst = jnp.concatenate([edge_index[1], pad]).reshape(ROWS, 128)
    dst64 = jnp.concatenate([edge_index[1], pad]).reshape(ROWS * 2, 64)
    zer1 = jnp.zeros((SPAN,), jnp.float32)
    zer2 = jnp.zeros((SPAN, 128), jnp.float32)

    degs_raw = _sc_degrees(src, dst, zer1)
    degs = jnp.transpose(degs_raw.reshape(2 * NC, NP))[:N]

    h = pl.pallas_call(
        _tc_norm_body,
        out_shape=jax.ShapeDtypeStruct((N, D), jnp.float32),
    )(x, degs)
    hp = jnp.concatenate([h, jnp.zeros((NP - N, D), jnp.float32)], axis=0)

    aggs = _sc_agg(hp, src, dst64, zer2)
    aggs = aggs[:, :N]

    out = pl.pallas_call(
        _tc_out_body,
        out_shape=jax.ShapeDtypeStruct((N, D + H), jnp.float32),
    )(x, degs, aggs, id_embedding, W_v, b_v, W_t, b_t)
    return out
